# Initial kernel scaffold; baseline (speedup 1.0000x reference)
import jax
import jax.numpy as jnp
from jax.experimental import pallas as pl


def kernel(sequence_bw, vocab_table, pos_table):
    raise NotImplementedError("write your pallas kernel here")


# speedup vs baseline: 1.2963x; 1.2963x over previous
import functools

import jax
import jax.numpy as jnp
from jax import lax
from jax.experimental import pallas as pl
from jax.experimental.pallas import tpu as pltpu
from jax.experimental.pallas import tpu_sc as plsc

_TOKENS = 1000000
_WORDS = 200
_COORDS = 32
_BATCH = 4096

_NW = 32
_ROWS = _BATCH * _WORDS
_ROWS_PER_W = _ROWS // _NW
_CHUNK = 512
_NCHUNK = _ROWS_PER_W // _CHUNK
_GATHER = 128
_NGATHER = _CHUNK // _GATHER
_POS_TILE = 4
_POS_ROWS = _POS_TILE * _WORDS


def _encoder(seq_flat, vocab_table, pos_table):
    mesh = plsc.VectorSubcoreMesh(core_axis_name="c", subcore_axis_name="s")

    @functools.partial(
        pl.kernel,
        mesh=mesh,
        out_type=jax.ShapeDtypeStruct((_ROWS, _COORDS), jnp.float32),
        scratch_types=[
            pltpu.VMEM((_CHUNK,), jnp.int32),
            pltpu.VMEM((_CHUNK, _COORDS), jnp.float32),
            pltpu.VMEM((_POS_ROWS, _COORDS), jnp.float32),
            pltpu.SemaphoreType.DMA,
        ],
        compiler_params=pltpu.CompilerParams(use_tc_tiling_on_sc=False),
    )
    def body(seq_hbm, vocab_hbm, pos_hbm, out_hbm, idx_v, rows_v, pos_v, sem):
        wid = lax.axis_index("s") * 2 + lax.axis_index("c")
        base = wid * _ROWS_PER_W

        for t in range(_POS_TILE):
            pltpu.sync_copy(pos_hbm, pos_v.at[pl.ds(t * _WORDS, _WORDS)])

        for c in range(_NCHUNK):
            r0 = base + c * _CHUNK
            p0 = (c * _CHUNK) % _WORDS
            pltpu.sync_copy(seq_hbm.at[pl.ds(r0, _CHUNK)], idx_v)
            copies = [
                pltpu.async_copy(
                    vocab_hbm.at[idx_v.at[pl.ds(j * _GATHER, _GATHER)]],
                    rows_v.at[pl.ds(j * _GATHER, _GATHER)],
                    sem,
                )
                for j in range(_NGATHER)
            ]
            for cp in copies:
                cp.wait()

            @pl.loop(0, _CHUNK)
            def _(r):
                pr = r + p0
                v0 = pos_v[pr, pl.ds(0, 16)]
                v1 = pos_v[pr, pl.ds(16, 16)]
                plsc.addupdate(rows_v.at[r, pl.ds(0, 16)], v0)
                plsc.addupdate(rows_v.at[r, pl.ds(16, 16)], v1)

            pltpu.sync_copy(rows_v, out_hbm.at[pl.ds(r0, _CHUNK)])

    return body(seq_flat, vocab_table, pos_table)


def kernel(sequence_bw, vocab_table, pos_table):
    seq_flat = sequence_bw.reshape(-1).astype(jnp.int32)
    out = _encoder(seq_flat, vocab_table, pos_table)
    return out.reshape(_BATCH, _WORDS, _COORDS)


# speedup vs baseline: 1.3748x; 1.0606x over previous
import functools

import jax
import jax.numpy as jnp
from jax import lax
from jax.experimental import pallas as pl
from jax.experimental.pallas import tpu as pltpu
from jax.experimental.pallas import tpu_sc as plsc

_TOKENS = 1000000
_WORDS = 200
_COORDS = 32
_BATCH = 4096

_NW = 32
_ROWS = _BATCH * _WORDS
_ROWS_PER_W = _ROWS // _NW
_CHUNK = 512
_NCHUNK = _ROWS_PER_W // _CHUNK
_GATHER = 128
_NGATHER = _CHUNK // _GATHER
_POS_TILE = 4
_POS_ROWS = _POS_TILE * _WORDS


def _encoder(seq_flat, vocab_table, pos_table):
    mesh = plsc.VectorSubcoreMesh(core_axis_name="c", subcore_axis_name="s")

    @functools.partial(
        pl.kernel,
        mesh=mesh,
        out_type=jax.ShapeDtypeStruct((_ROWS, _COORDS), jnp.float32),
        scratch_types=[
            pltpu.VMEM((2, _CHUNK), jnp.int32),
            pltpu.VMEM((2, _CHUNK, _COORDS), jnp.float32),
            pltpu.VMEM((_POS_ROWS, _COORDS), jnp.float32),
            pltpu.SemaphoreType.DMA,
            pltpu.SemaphoreType.DMA,
            pltpu.SemaphoreType.DMA,
            pltpu.SemaphoreType.DMA,
        ],
        compiler_params=pltpu.CompilerParams(use_tc_tiling_on_sc=False),
    )
    def body(seq_hbm, vocab_hbm, pos_hbm, out_hbm, idx_v, rows_v, pos_v,
             gsem0, gsem1, osem0, osem1):
        wid = lax.axis_index("s") * 2 + lax.axis_index("c")
        base = wid * _ROWS_PER_W
        gsems = (gsem0, gsem1)
        osems = (osem0, osem1)

        for t in range(_POS_TILE):
            pltpu.sync_copy(pos_hbm, pos_v.at[pl.ds(t * _WORDS, _WORDS)])

        def fire(c):
            buf = c % 2
            r0 = base + c * _CHUNK
            pltpu.sync_copy(seq_hbm.at[pl.ds(r0, _CHUNK)], idx_v.at[buf])
            return [
                pltpu.async_copy(
                    vocab_hbm.at[idx_v.at[buf, pl.ds(j * _GATHER, _GATHER)]],
                    rows_v.at[buf, pl.ds(j * _GATHER, _GATHER)],
                    gsems[buf],
                )
                for j in range(_NGATHER)
            ]

        gathers = {0: fire(0)}
        writebacks = {}
        for c in range(_NCHUNK):
            buf = c % 2
            if c + 1 < _NCHUNK:
                if c - 1 in writebacks:
                    writebacks.pop(c - 1).wait()
                gathers[c + 1] = fire(c + 1)
            for cp in gathers.pop(c):
                cp.wait()

            p0 = (c * _CHUNK) % _WORDS

            @pl.loop(0, _CHUNK)
            def _(r):
                pr = r + p0
                v0 = pos_v[pr, pl.ds(0, 16)]
                v1 = pos_v[pr, pl.ds(16, 16)]
                plsc.addupdate(rows_v.at[buf, r, pl.ds(0, 16)], v0)
                plsc.addupdate(rows_v.at[buf, r, pl.ds(16, 16)], v1)

            writebacks[c] = pltpu.async_copy(
                rows_v.at[buf], out_hbm.at[pl.ds(base + c * _CHUNK, _CHUNK)],
                osems[buf],
            )
        for wb in writebacks.values():
            wb.wait()

    return body(seq_flat, vocab_table, pos_table)


def kernel(sequence_bw, vocab_table, pos_table):
    seq_flat = sequence_bw.reshape(-1).astype(jnp.int32)
    out = _encoder(seq_flat, vocab_table, pos_table)
    return out.reshape(_BATCH, _WORDS, _COORDS)


# speedup vs baseline: 1.4585x; 1.0609x over previous
import functools

import jax
import jax.numpy as jnp
from jax import lax
from jax.experimental import pallas as pl
from jax.experimental.pallas import tpu as pltpu
from jax.experimental.pallas import tpu_sc as plsc

_TOKENS = 1000000
_WORDS = 200
_COORDS = 32
_BATCH = 4096

_NW = 32
_ROWS = _BATCH * _WORDS
_ROWS_PER_W = _ROWS // _NW
_CHUNK = 512
_NCHUNK = _ROWS_PER_W // _CHUNK
_GATHER = 128
_NGATHER = _CHUNK // _GATHER
_POS_TILE = 4
_POS_ROWS = _POS_TILE * _WORDS
_NBUF = 4
_LOOKAHEAD = 2


def _encoder(seq_flat, vocab_table, pos_table):
    mesh = plsc.VectorSubcoreMesh(core_axis_name="c", subcore_axis_name="s")

    @functools.partial(
        pl.kernel,
        mesh=mesh,
        out_type=jax.ShapeDtypeStruct((_ROWS, _COORDS), jnp.float32),
        scratch_types=[
            pltpu.VMEM((_ROWS_PER_W,), jnp.int32),
            pltpu.VMEM((_NBUF, _CHUNK, _COORDS), jnp.float32),
            pltpu.VMEM((_POS_ROWS, _COORDS), jnp.float32),
        ]
        + [pltpu.SemaphoreType.DMA] * (2 * _NBUF),
        compiler_params=pltpu.CompilerParams(use_tc_tiling_on_sc=False),
    )
    def body(seq_hbm, vocab_hbm, pos_hbm, out_hbm, idx_v, rows_v, pos_v,
             *sems):
        wid = lax.axis_index("s") * 2 + lax.axis_index("c")
        base = wid * _ROWS_PER_W
        gsems = sems[:_NBUF]
        osems = sems[_NBUF:]

        pltpu.sync_copy(seq_hbm.at[pl.ds(base, _ROWS_PER_W)], idx_v)

        for t in range(_POS_TILE):
            pltpu.sync_copy(pos_hbm, pos_v.at[pl.ds(t * _WORDS, _WORDS)])

        def fire(c):
            buf = c % _NBUF
            return [
                pltpu.async_copy(
                    vocab_hbm.at[
                        idx_v.at[pl.ds(c * _CHUNK + j * _GATHER, _GATHER)]],
                    rows_v.at[buf, pl.ds(j * _GATHER, _GATHER)],
                    gsems[buf],
                )
                for j in range(_NGATHER)
            ]

        gathers = {c: fire(c) for c in range(min(_LOOKAHEAD, _NCHUNK))}
        writebacks = {}
        for c in range(_NCHUNK):
            buf = c % _NBUF
            nxt = c + _LOOKAHEAD
            if nxt < _NCHUNK:
                if nxt - _NBUF in writebacks:
                    writebacks.pop(nxt - _NBUF).wait()
                gathers[nxt] = fire(nxt)
            for cp in gathers.pop(c):
                cp.wait()

            p0 = (c * _CHUNK) % _WORDS

            @pl.loop(0, _CHUNK)
            def _(r):
                pr = r + p0
                v0 = pos_v[pr, pl.ds(0, 16)]
                v1 = pos_v[pr, pl.ds(16, 16)]
                plsc.addupdate(rows_v.at[buf, r, pl.ds(0, 16)], v0)
                plsc.addupdate(rows_v.at[buf, r, pl.ds(16, 16)], v1)

            writebacks[c] = pltpu.async_copy(
                rows_v.at[buf], out_hbm.at[pl.ds(base + c * _CHUNK, _CHUNK)],
                osems[buf],
            )
        for wb in writebacks.values():
            wb.wait()

    return body(seq_flat, vocab_table, pos_table)


def kernel(sequence_bw, vocab_table, pos_table):
    seq_flat = sequence_bw.reshape(-1).astype(jnp.int32)
    out = _encoder(seq_flat, vocab_table, pos_table)
    return out.reshape(_BATCH, _WORDS, _COORDS)
